# baseline (speedup 1.0000x reference)
import jax
import jax.numpy as jnp
from jax import lax
from jax.experimental import pallas as pl
from jax.experimental.pallas import tpu as pltpu
from jax.experimental.pallas import tpu_sc as plsc

N = 10000
E = 320000
D_IN = 128
D_EDGE = 16
OUT = 64
HEADS = 2

NC = 2
NS = 16
NW = NC * NS
NP = 10240
CH = 40
RPT = NP // NS
PH = 16
EH = 163840


def _mm2_body(a_ref, w1_ref, w2_ref, o1_ref, o2_ref):
    a = a_ref[...]
    o1_ref[...] = jnp.dot(a, w1_ref[...], preferred_element_type=jnp.float32)
    o2_ref[...] = jnp.dot(a, w2_ref[...], preferred_element_type=jnp.float32)


def _mm2(a, w1, w2, block_m):
    m, k = a.shape
    grid = (m // block_m,)
    return pl.pallas_call(
        _mm2_body,
        grid=grid,
        in_specs=[
            pl.BlockSpec((block_m, k), lambda i: (i, 0)),
            pl.BlockSpec(w1.shape, lambda i: (0, 0)),
            pl.BlockSpec(w2.shape, lambda i: (0, 0)),
        ],
        out_specs=[
            pl.BlockSpec((block_m, w1.shape[1]), lambda i: (i, 0)),
            pl.BlockSpec((block_m, w2.shape[1]), lambda i: (i, 0)),
        ],
        out_shape=[
            jax.ShapeDtypeStruct((m, w1.shape[1]), jnp.float32),
            jax.ShapeDtypeStruct((m, w2.shape[1]), jnp.float32),
        ],
    )(a, w1, w2)


def _edge_tc_body(heads, ch, gxl_ref, gxr_ref, eat_ref, we_ref, am_ref,
                  ex_ref, msg_ref):
    gxl = gxl_ref[...]
    ea = jnp.dot(eat_ref[...], we_ref[...],
                 preferred_element_type=jnp.float32)
    z = gxl + gxr_ref[...] + ea
    z = jnp.where(z > 0, z, 0.2 * z)
    lg = jnp.dot(z, am_ref[...], preferred_element_type=jnp.float32)
    ex = jnp.exp(jnp.clip(lg, -75.0, 75.0))
    bm = ex.shape[0]
    ex_ref[...] = jnp.concatenate(
        [ex, jnp.zeros((bm, PH - heads), jnp.float32)], axis=1)
    for h in range(heads):
        sl = slice(h * ch, (h + 1) * ch)
        msg_ref[:, sl] = gxl[:, sl] * ex[:, h:h + 1]


def _edge_tc(gxl, gxr, edge_attr, we, att_mat, heads, block_m):
    m, w = gxl.shape
    ch = w // heads
    grid = (m // block_m,)
    import functools
    return pl.pallas_call(
        functools.partial(_edge_tc_body, heads, ch),
        grid=grid,
        in_specs=[
            pl.BlockSpec((block_m, w), lambda i: (i, 0)),
            pl.BlockSpec((block_m, w), lambda i: (i, 0)),
            pl.BlockSpec((block_m, D_EDGE), lambda i: (i, 0)),
            pl.BlockSpec((D_EDGE, w), lambda i: (0, 0)),
            pl.BlockSpec((w, heads), lambda i: (0, 0)),
        ],
        out_specs=[
            pl.BlockSpec((block_m, PH), lambda i: (i, 0)),
            pl.BlockSpec((block_m, w), lambda i: (i, 0)),
        ],
        out_shape=[
            jax.ShapeDtypeStruct((m, PH), jnp.float32),
            jax.ShapeDtypeStruct((m, w), jnp.float32),
        ],
    )(gxl, gxr, edge_attr, we, att_mat)


def _combine_mm2_body(ch, heads, a0_ref, a1_ref, a2_ref, a3_ref, d0_ref,
                      d1_ref, d2_ref, d3_ref, b_ref, w1_ref, w2_ref, o1_ref,
                      o2_ref):
    x = a0_ref[...] + a1_ref[...] + a2_ref[...] + a3_ref[...]
    den = d0_ref[...] + d1_ref[...] + d2_ref[...] + d3_ref[...] + 1e-16
    cols = []
    for h in range(heads):
        cols.append(x[:, h * ch:(h + 1) * ch] / den[:, h:h + 1])
    x = jnp.concatenate(cols, axis=1) if heads > 1 else cols[0]
    x = jnp.maximum(x + b_ref[...], 0.0)
    o1_ref[...] = jnp.dot(x, w1_ref[...], preferred_element_type=jnp.float32)
    o2_ref[...] = jnp.dot(x, w2_ref[...], preferred_element_type=jnp.float32)


def _combine_mm2(aa, dd, b, w1, w2, heads, block_m):
    m, w = aa[0].shape
    ch = w // heads
    grid = (m // block_m,)
    import functools
    return pl.pallas_call(
        functools.partial(_combine_mm2_body, ch, heads),
        grid=grid,
        in_specs=[
            pl.BlockSpec((block_m, w), lambda i: (i, 0)),
            pl.BlockSpec((block_m, w), lambda i: (i, 0)),
            pl.BlockSpec((block_m, w), lambda i: (i, 0)),
            pl.BlockSpec((block_m, w), lambda i: (i, 0)),
            pl.BlockSpec((block_m, PH), lambda i: (i, 0)),
            pl.BlockSpec((block_m, PH), lambda i: (i, 0)),
            pl.BlockSpec((block_m, PH), lambda i: (i, 0)),
            pl.BlockSpec((block_m, PH), lambda i: (i, 0)),
            pl.BlockSpec((1, w), lambda i: (0, 0)),
            pl.BlockSpec(w1.shape, lambda i: (0, 0)),
            pl.BlockSpec(w2.shape, lambda i: (0, 0)),
        ],
        out_specs=[
            pl.BlockSpec((block_m, w1.shape[1]), lambda i: (i, 0)),
            pl.BlockSpec((block_m, w2.shape[1]), lambda i: (i, 0)),
        ],
        out_shape=[
            jax.ShapeDtypeStruct((m, w1.shape[1]), jnp.float32),
            jax.ShapeDtypeStruct((m, w2.shape[1]), jnp.float32),
        ],
    )(*aa, *dd, b.reshape(1, w), w1, w2)


def _combine_final_body(ow, a0_ref, a1_ref, a2_ref, a3_ref, d0_ref, d1_ref,
                        d2_ref, d3_ref, b_ref, o_ref):
    x = a0_ref[...] + a1_ref[...] + a2_ref[...] + a3_ref[...]
    den = d0_ref[...] + d1_ref[...] + d2_ref[...] + d3_ref[...] + 1e-16
    o_ref[...] = jnp.maximum(x[:, :ow] / den[:, 0:1] + b_ref[...], 0.0)


def _combine_final(aa, dd, b, block_m):
    m, w = aa[0].shape
    ow = b.shape[0]
    grid = (m // block_m,)
    import functools
    return pl.pallas_call(
        functools.partial(_combine_final_body, ow),
        grid=grid,
        in_specs=[
            pl.BlockSpec((block_m, w), lambda i: (i, 0)),
            pl.BlockSpec((block_m, w), lambda i: (i, 0)),
            pl.BlockSpec((block_m, w), lambda i: (i, 0)),
            pl.BlockSpec((block_m, w), lambda i: (i, 0)),
            pl.BlockSpec((block_m, PH), lambda i: (i, 0)),
            pl.BlockSpec((block_m, PH), lambda i: (i, 0)),
            pl.BlockSpec((block_m, PH), lambda i: (i, 0)),
            pl.BlockSpec((block_m, PH), lambda i: (i, 0)),
            pl.BlockSpec((1, ow), lambda i: (0, 0)),
        ],
        out_specs=pl.BlockSpec((block_m, ow), lambda i: (i, 0)),
        out_shape=jax.ShapeDtypeStruct((m, ow), jnp.float32),
    )(*aa, *dd, b.reshape(1, ow))


def _sc_mesh():
    return plsc.VectorSubcoreMesh(core_axis_name="c", subcore_axis_name="s",
                                  num_cores=NC, num_subcores=NS)


def _sc_gather(xl, xr, src, dst, *, width):
    rows = src.shape[0]
    ept = rows // NW
    ngrp = ept // (2 * CH)

    def body(xl_hbm, xr_hbm, src_hbm, dst_hbm, gxl_hbm, gxr_hbm,
             src0, dst0, xl0, xr0, src1, dst1, xl1, xr1,
             si1, si2, sg1, sg2, st1, st2):
        c = lax.axis_index("c")
        s = lax.axis_index("s")
        wid = c * NS + s

        def grp(m, _):
            ba = wid * ept + (2 * m) * CH
            bb = ba + CH
            ia1 = pltpu.async_copy(src_hbm.at[pl.ds(ba, CH)], src0, si1)
            ia2 = pltpu.async_copy(dst_hbm.at[pl.ds(ba, CH)], dst0, si2)
            ia1.wait()
            ia2.wait()
            ga1 = pltpu.async_copy(xl_hbm.at[src0], xl0, sg1)
            ga2 = pltpu.async_copy(xr_hbm.at[dst0], xr0, sg2)
            ib1 = pltpu.async_copy(src_hbm.at[pl.ds(bb, CH)], src1, si1)
            ib2 = pltpu.async_copy(dst_hbm.at[pl.ds(bb, CH)], dst1, si2)
            ga1.wait()
            ga2.wait()
            ib1.wait()
            ib2.wait()
            gb1 = pltpu.async_copy(xl_hbm.at[src1], xl1, sg1)
            gb2 = pltpu.async_copy(xr_hbm.at[dst1], xr1, sg2)
            sa1 = pltpu.async_copy(xl0, gxl_hbm.at[pl.ds(ba, CH)], st1)
            sa2 = pltpu.async_copy(xr0, gxr_hbm.at[pl.ds(ba, CH)], st2)
            gb1.wait()
            gb2.wait()
            sb1 = pltpu.async_copy(xl1, gxl_hbm.at[pl.ds(bb, CH)], st1)
            sb2 = pltpu.async_copy(xr1, gxr_hbm.at[pl.ds(bb, CH)], st2)
            sa1.wait()
            sa2.wait()
            sb1.wait()
            sb2.wait()
            return 0

        lax.fori_loop(0, ngrp, grp, 0)

    f = pl.kernel(
        body,
        out_type=[
            jax.ShapeDtypeStruct((rows, width), jnp.float32),
            jax.ShapeDtypeStruct((rows, width), jnp.float32),
        ],
        mesh=_sc_mesh(),
        scratch_types=[
            pltpu.VMEM((CH,), jnp.int32),
            pltpu.VMEM((CH,), jnp.int32),
            pltpu.VMEM((CH, width), jnp.float32),
            pltpu.VMEM((CH, width), jnp.float32),
            pltpu.VMEM((CH,), jnp.int32),
            pltpu.VMEM((CH,), jnp.int32),
            pltpu.VMEM((CH, width), jnp.float32),
            pltpu.VMEM((CH, width), jnp.float32),
            pltpu.SemaphoreType.DMA,
            pltpu.SemaphoreType.DMA,
            pltpu.SemaphoreType.DMA,
            pltpu.SemaphoreType.DMA,
            pltpu.SemaphoreType.DMA,
            pltpu.SemaphoreType.DMA,
        ],
    )
    return f(xl, xr, src, dst)


def _sc_scatter(ex, msg, dst, *, heads, width):
    rows = dst.shape[0]
    ept = rows // NW
    ngrp = ept // (2 * CH)

    def body(ex_hbm, msg_hbm, dst_hbm, den_hbm, acc_hbm,
             dst0, ex0, msg0, dst1, ex1, msg1, den_sh, acc_sh,
             sl1, sl2, sl3, sc1, sc2):
        c = lax.axis_index("c")
        s = lax.axis_index("s")
        wid = c * NS + s
        row0 = s * RPT
        zv = jnp.zeros((16,), jnp.float32)

        def zrow(i, _):
            ex0[i, pl.ds(0, PH)] = zv
            for v in range(width // 16):
                msg0[i, pl.ds(v * 16, 16)] = zv
            return 0

        lax.fori_loop(0, CH, zrow, 0)
        for q in range(RPT // CH):
            pltpu.sync_copy(msg0, acc_sh.at[pl.ds(row0 + q * CH, CH)])
            pltpu.sync_copy(ex0, den_sh.at[pl.ds(row0 + q * CH, CH)])
        plsc.subcore_barrier()

        def grp(m, _):
            ba = wid * ept + (2 * m) * CH
            bb = ba + CH
            la1 = pltpu.async_copy(dst_hbm.at[pl.ds(ba, CH)], dst0, sl1)
            la2 = pltpu.async_copy(ex_hbm.at[pl.ds(ba, CH)], ex0, sl2)
            la3 = pltpu.async_copy(msg_hbm.at[pl.ds(ba, CH)], msg0, sl3)
            lb1 = pltpu.async_copy(dst_hbm.at[pl.ds(bb, CH)], dst1, sl1)
            lb2 = pltpu.async_copy(ex_hbm.at[pl.ds(bb, CH)], ex1, sl2)
            lb3 = pltpu.async_copy(msg_hbm.at[pl.ds(bb, CH)], msg1, sl3)
            la1.wait()
            la2.wait()
            la3.wait()
            ca1 = pltpu.async_copy(ex0, den_sh.at[dst0], sc1, add=True)
            ca2 = pltpu.async_copy(msg0, acc_sh.at[dst0], sc2, add=True)
            lb1.wait()
            lb2.wait()
            lb3.wait()
            ca1.wait()
            ca2.wait()
            cb1 = pltpu.async_copy(ex1, den_sh.at[dst1], sc1, add=True)
            cb2 = pltpu.async_copy(msg1, acc_sh.at[dst1], sc2, add=True)
            cb1.wait()
            cb2.wait()
            return 0

        lax.fori_loop(0, ngrp, grp, 0)
        plsc.subcore_barrier()
        pltpu.sync_copy(den_sh.at[pl.ds(row0, RPT)],
                        den_hbm.at[c, pl.ds(row0, RPT)])
        pltpu.sync_copy(acc_sh.at[pl.ds(row0, RPT)],
                        acc_hbm.at[c, pl.ds(row0, RPT)])

    f = pl.kernel(
        body,
        out_type=[
            jax.ShapeDtypeStruct((NC, NP, PH), jnp.float32),
            jax.ShapeDtypeStruct((NC, NP, width), jnp.float32),
        ],
        mesh=_sc_mesh(),
        scratch_types=[
            pltpu.VMEM((CH,), jnp.int32),
            pltpu.VMEM((CH, PH), jnp.float32),
            pltpu.VMEM((CH, width), jnp.float32),
            pltpu.VMEM((CH,), jnp.int32),
            pltpu.VMEM((CH, PH), jnp.float32),
            pltpu.VMEM((CH, width), jnp.float32),
            pltpu.VMEM_SHARED((NP, PH), jnp.float32),
            pltpu.VMEM_SHARED((NP, width), jnp.float32),
            pltpu.SemaphoreType.DMA,
            pltpu.SemaphoreType.DMA,
            pltpu.SemaphoreType.DMA,
            pltpu.SemaphoreType.DMA,
            pltpu.SemaphoreType.DMA,
        ],
    )
    return f(ex, msg, dst)


def _gat_edge_pass(xlp, xrp, ea_h, src_h, dst_h, we, att_mat, *, heads,
                   width):
    accs, dens = [], []
    exs, msgs = [], []
    for h in range(2):
        gxl, gxr = _sc_gather(xlp, xrp, src_h[h], dst_h[h], width=width)
        ex, msg = _edge_tc(gxl, gxr, ea_h[h], we, att_mat, heads,
                           block_m=2048)
        exs.append(ex)
        msgs.append(msg)
    for h in range(2):
        den_p, acc_p = _sc_scatter(exs[h], msgs[h], dst_h[h], heads=heads,
                                   width=width)
        accs.extend([acc_p[0, :N], acc_p[1, :N]])
        dens.extend([den_p[0, :N], den_p[1, :N]])
    return accs, dens


def _pad_half(arr, fill, rows):
    pad = EH - rows
    shp = (pad,) + arr.shape[1:]
    return jnp.concatenate([arr, jnp.full(shp, fill, arr.dtype)], axis=0)


def kernel(node_fts, edge_index, edge_attr, Wl1, Wr1, We1, att1, b1,
           Wl2, Wr2, We2, att2, b2):
    src = edge_index[0]
    dst = edge_index[1]

    e2 = E // 2
    src_h = [_pad_half(src[:e2], 0, e2), _pad_half(src[e2:], 0, e2)]
    dst_h = [_pad_half(dst[:e2], N, e2), _pad_half(dst[e2:], N, e2)]
    ea_h = [_pad_half(edge_attr[:e2], 0.0, e2),
            _pad_half(edge_attr[e2:], 0.0, e2)]
    npad = jnp.zeros((NP - N, HEADS * OUT), jnp.float32)

    z64 = jnp.zeros((OUT, 1), jnp.float32)
    am1 = jnp.concatenate([
        jnp.concatenate([att1[0][:, None], z64], axis=1),
        jnp.concatenate([z64, att1[1][:, None]], axis=1),
    ], axis=0)
    am2 = jnp.concatenate([att2.T, z64], axis=0)
    zp = jnp.zeros((HEADS * OUT, OUT), jnp.float32)
    Wl2p = jnp.concatenate([Wl2, zp], axis=1)
    Wr2p = jnp.concatenate([Wr2, zp], axis=1)
    We2p = jnp.concatenate([We2, jnp.zeros((D_EDGE, OUT), jnp.float32)],
                           axis=1)

    xl1, xr1 = _mm2(node_fts, Wl1, Wr1, block_m=1000)
    xl1p = jnp.concatenate([xl1, npad], axis=0)
    xr1p = jnp.concatenate([xr1, npad], axis=0)

    acc1, den1 = _gat_edge_pass(xl1p, xr1p, ea_h, src_h, dst_h, We1, am1,
                                heads=HEADS, width=HEADS * OUT)

    xl2, xr2 = _combine_mm2(acc1, den1, b1, Wl2p, Wr2p, heads=HEADS,
                            block_m=1000)
    xl2p = jnp.concatenate([xl2, npad], axis=0)
    xr2p = jnp.concatenate([xr2, npad], axis=0)

    acc2, den2 = _gat_edge_pass(xl2p, xr2p, ea_h, src_h, dst_h, We2p, am2,
                                heads=1, width=HEADS * OUT)

    return _combine_final(acc2, den2, b2, block_m=1000)


# speedup vs baseline: 1.4626x; 1.4626x over previous
import jax
import jax.numpy as jnp
from jax import lax
from jax.experimental import pallas as pl
from jax.experimental.pallas import tpu as pltpu
from jax.experimental.pallas import tpu_sc as plsc

N = 10000
E = 320000
D_IN = 128
D_EDGE = 16
OUT = 64
HEADS = 2

NC = 2
NS = 16
NW = NC * NS
NP = 10240
EPT = E // NW
CH = 40
NGRP = EPT // (2 * CH)
RPT = NP // NS
PH = 16


def _mm2_body(a_ref, w1_ref, w2_ref, o1_ref, o2_ref):
    a = a_ref[...]
    o1_ref[...] = jnp.dot(a, w1_ref[...], preferred_element_type=jnp.float32)
    o2_ref[...] = jnp.dot(a, w2_ref[...], preferred_element_type=jnp.float32)


def _mm2(a, w1, w2, block_m):
    m, k = a.shape
    grid = (m // block_m,)
    return pl.pallas_call(
        _mm2_body,
        grid=grid,
        in_specs=[
            pl.BlockSpec((block_m, k), lambda i: (i, 0)),
            pl.BlockSpec(w1.shape, lambda i: (0, 0)),
            pl.BlockSpec(w2.shape, lambda i: (0, 0)),
        ],
        out_specs=[
            pl.BlockSpec((block_m, w1.shape[1]), lambda i: (i, 0)),
            pl.BlockSpec((block_m, w2.shape[1]), lambda i: (i, 0)),
        ],
        out_shape=[
            jax.ShapeDtypeStruct((m, w1.shape[1]), jnp.float32),
            jax.ShapeDtypeStruct((m, w2.shape[1]), jnp.float32),
        ],
    )(a, w1, w2)


def _edge_tc_body(heads, ch, gxl_ref, gxr_ref, eat_ref, we_ref, am_ref,
                  ex_ref, msg_ref):
    gxl = gxl_ref[...]
    ea = jnp.dot(eat_ref[...], we_ref[...],
                 preferred_element_type=jnp.float32)
    z = gxl + gxr_ref[...] + ea
    z = jnp.where(z > 0, z, 0.2 * z)
    lg = jnp.dot(z, am_ref[...], preferred_element_type=jnp.float32)
    ex = jnp.exp(jnp.clip(lg, -75.0, 75.0))
    bm = ex.shape[0]
    ex_ref[...] = jnp.concatenate(
        [ex, jnp.zeros((bm, PH - heads), jnp.float32)], axis=1)
    for h in range(heads):
        sl = slice(h * ch, (h + 1) * ch)
        msg_ref[:, sl] = gxl[:, sl] * ex[:, h:h + 1]


def _edge_tc(gxl, gxr, edge_attr, we, att_mat, heads, block_m):
    m, w = gxl.shape
    ch = w // heads
    grid = (m // block_m,)
    import functools
    return pl.pallas_call(
        functools.partial(_edge_tc_body, heads, ch),
        grid=grid,
        in_specs=[
            pl.BlockSpec((block_m, w), lambda i: (i, 0)),
            pl.BlockSpec((block_m, w), lambda i: (i, 0)),
            pl.BlockSpec((block_m, D_EDGE), lambda i: (i, 0)),
            pl.BlockSpec((D_EDGE, w), lambda i: (0, 0)),
            pl.BlockSpec((w, heads), lambda i: (0, 0)),
        ],
        out_specs=[
            pl.BlockSpec((block_m, PH), lambda i: (i, 0)),
            pl.BlockSpec((block_m, w), lambda i: (i, 0)),
        ],
        out_shape=[
            jax.ShapeDtypeStruct((m, PH), jnp.float32),
            jax.ShapeDtypeStruct((m, w), jnp.float32),
        ],
    )(gxl, gxr, edge_attr, we, att_mat)


def _combine_mm2_body(ch, heads, a0_ref, a1_ref, d0_ref, d1_ref, b_ref,
                      w1_ref, w2_ref, o1_ref, o2_ref):
    x = a0_ref[...] + a1_ref[...]
    den = d0_ref[...] + d1_ref[...] + 1e-16
    cols = []
    for h in range(heads):
        cols.append(x[:, h * ch:(h + 1) * ch] / den[:, h:h + 1])
    x = jnp.concatenate(cols, axis=1) if heads > 1 else cols[0]
    x = jnp.maximum(x + b_ref[...], 0.0)
    o1_ref[...] = jnp.dot(x, w1_ref[...], preferred_element_type=jnp.float32)
    o2_ref[...] = jnp.dot(x, w2_ref[...], preferred_element_type=jnp.float32)


def _combine_mm2(a0, a1, d0, d1, b, w1, w2, heads, block_m):
    m, w = a0.shape
    ch = w // heads
    grid = (m // block_m,)
    import functools
    return pl.pallas_call(
        functools.partial(_combine_mm2_body, ch, heads),
        grid=grid,
        in_specs=[
            pl.BlockSpec((block_m, w), lambda i: (i, 0)),
            pl.BlockSpec((block_m, w), lambda i: (i, 0)),
            pl.BlockSpec((block_m, PH), lambda i: (i, 0)),
            pl.BlockSpec((block_m, PH), lambda i: (i, 0)),
            pl.BlockSpec((1, w), lambda i: (0, 0)),
            pl.BlockSpec(w1.shape, lambda i: (0, 0)),
            pl.BlockSpec(w2.shape, lambda i: (0, 0)),
        ],
        out_specs=[
            pl.BlockSpec((block_m, w1.shape[1]), lambda i: (i, 0)),
            pl.BlockSpec((block_m, w2.shape[1]), lambda i: (i, 0)),
        ],
        out_shape=[
            jax.ShapeDtypeStruct((m, w1.shape[1]), jnp.float32),
            jax.ShapeDtypeStruct((m, w2.shape[1]), jnp.float32),
        ],
    )(a0, a1, d0, d1, b.reshape(1, w), w1, w2)


def _combine_final_body(ow, a0_ref, a1_ref, d0_ref, d1_ref, b_ref, o_ref):
    x = a0_ref[...] + a1_ref[...]
    den = d0_ref[...] + d1_ref[...] + 1e-16
    o_ref[...] = jnp.maximum(x[:, :ow] / den[:, 0:1] + b_ref[...], 0.0)


def _combine_final(a0, a1, d0, d1, b, block_m):
    m, w = a0.shape
    ow = b.shape[0]
    grid = (m // block_m,)
    import functools
    return pl.pallas_call(
        functools.partial(_combine_final_body, ow),
        grid=grid,
        in_specs=[
            pl.BlockSpec((block_m, w), lambda i: (i, 0)),
            pl.BlockSpec((block_m, w), lambda i: (i, 0)),
            pl.BlockSpec((block_m, PH), lambda i: (i, 0)),
            pl.BlockSpec((block_m, PH), lambda i: (i, 0)),
            pl.BlockSpec((1, ow), lambda i: (0, 0)),
        ],
        out_specs=pl.BlockSpec((block_m, ow), lambda i: (i, 0)),
        out_shape=jax.ShapeDtypeStruct((m, ow), jnp.float32),
    )(a0, a1, d0, d1, b.reshape(1, ow))


def _sc_mesh():
    return plsc.VectorSubcoreMesh(core_axis_name="c", subcore_axis_name="s",
                                  num_cores=NC, num_subcores=NS)


def _sc_gather(xl, xr, src, dst, *, width):
    def body(xl_hbm, xr_hbm, src_hbm, dst_hbm, gxl_hbm, gxr_hbm,
             src0, dst0, xl0, xr0, src1, dst1, xl1, xr1,
             si1, si2, sg1, sg2, st1, st2):
        c = lax.axis_index("c")
        s = lax.axis_index("s")
        wid = c * NS + s

        def grp(m, _):
            ba = wid * EPT + (2 * m) * CH
            bb = ba + CH
            ia1 = pltpu.async_copy(src_hbm.at[pl.ds(ba, CH)], src0, si1)
            ia2 = pltpu.async_copy(dst_hbm.at[pl.ds(ba, CH)], dst0, si2)
            ia1.wait()
            ia2.wait()
            ga1 = pltpu.async_copy(xl_hbm.at[src0], xl0, sg1)
            ga2 = pltpu.async_copy(xr_hbm.at[dst0], xr0, sg2)
            ib1 = pltpu.async_copy(src_hbm.at[pl.ds(bb, CH)], src1, si1)
            ib2 = pltpu.async_copy(dst_hbm.at[pl.ds(bb, CH)], dst1, si2)
            ga1.wait()
            ga2.wait()
            ib1.wait()
            ib2.wait()
            gb1 = pltpu.async_copy(xl_hbm.at[src1], xl1, sg1)
            gb2 = pltpu.async_copy(xr_hbm.at[dst1], xr1, sg2)
            sa1 = pltpu.async_copy(xl0, gxl_hbm.at[pl.ds(ba, CH)], st1)
            sa2 = pltpu.async_copy(xr0, gxr_hbm.at[pl.ds(ba, CH)], st2)
            gb1.wait()
            gb2.wait()
            sb1 = pltpu.async_copy(xl1, gxl_hbm.at[pl.ds(bb, CH)], st1)
            sb2 = pltpu.async_copy(xr1, gxr_hbm.at[pl.ds(bb, CH)], st2)
            sa1.wait()
            sa2.wait()
            sb1.wait()
            sb2.wait()
            return 0

        lax.fori_loop(0, NGRP, grp, 0)

    f = pl.kernel(
        body,
        out_type=[
            jax.ShapeDtypeStruct((E, width), jnp.float32),
            jax.ShapeDtypeStruct((E, width), jnp.float32),
        ],
        mesh=_sc_mesh(),
        scratch_types=[
            pltpu.VMEM((CH,), jnp.int32),
            pltpu.VMEM((CH,), jnp.int32),
            pltpu.VMEM((CH, width), jnp.float32),
            pltpu.VMEM((CH, width), jnp.float32),
            pltpu.VMEM((CH,), jnp.int32),
            pltpu.VMEM((CH,), jnp.int32),
            pltpu.VMEM((CH, width), jnp.float32),
            pltpu.VMEM((CH, width), jnp.float32),
            pltpu.SemaphoreType.DMA,
            pltpu.SemaphoreType.DMA,
            pltpu.SemaphoreType.DMA,
            pltpu.SemaphoreType.DMA,
            pltpu.SemaphoreType.DMA,
            pltpu.SemaphoreType.DMA,
        ],
    )
    return f(xl, xr, src, dst)


def _sc_scatter(ex, msg, dst, *, heads, width):
    def body(ex_hbm, msg_hbm, dst_hbm, den_hbm, acc_hbm,
             dst0, ex0, msg0, dst1, ex1, msg1, den_sh, acc_sh,
             sl1, sl2, sl3, sc1, sc2):
        c = lax.axis_index("c")
        s = lax.axis_index("s")
        wid = c * NS + s
        row0 = s * RPT
        zv = jnp.zeros((16,), jnp.float32)

        def zrow(i, _):
            ex0[i, pl.ds(0, PH)] = zv
            for v in range(width // 16):
                msg0[i, pl.ds(v * 16, 16)] = zv
            return 0

        lax.fori_loop(0, CH, zrow, 0)
        for q in range(RPT // CH):
            pltpu.sync_copy(msg0, acc_sh.at[pl.ds(row0 + q * CH, CH)])
            pltpu.sync_copy(ex0, den_sh.at[pl.ds(row0 + q * CH, CH)])
        plsc.subcore_barrier()

        def grp(m, _):
            ba = wid * EPT + (2 * m) * CH
            bb = ba + CH
            la1 = pltpu.async_copy(dst_hbm.at[pl.ds(ba, CH)], dst0, sl1)
            la2 = pltpu.async_copy(ex_hbm.at[pl.ds(ba, CH)], ex0, sl2)
            la3 = pltpu.async_copy(msg_hbm.at[pl.ds(ba, CH)], msg0, sl3)
            lb1 = pltpu.async_copy(dst_hbm.at[pl.ds(bb, CH)], dst1, sl1)
            lb2 = pltpu.async_copy(ex_hbm.at[pl.ds(bb, CH)], ex1, sl2)
            lb3 = pltpu.async_copy(msg_hbm.at[pl.ds(bb, CH)], msg1, sl3)
            la1.wait()
            la2.wait()
            la3.wait()
            ca1 = pltpu.async_copy(ex0, den_sh.at[dst0], sc1, add=True)
            ca2 = pltpu.async_copy(msg0, acc_sh.at[dst0], sc2, add=True)
            lb1.wait()
            lb2.wait()
            lb3.wait()
            ca1.wait()
            ca2.wait()
            cb1 = pltpu.async_copy(ex1, den_sh.at[dst1], sc1, add=True)
            cb2 = pltpu.async_copy(msg1, acc_sh.at[dst1], sc2, add=True)
            cb1.wait()
            cb2.wait()
            return 0

        lax.fori_loop(0, NGRP, grp, 0)
        plsc.subcore_barrier()
        pltpu.sync_copy(den_sh.at[pl.ds(row0, RPT)],
                        den_hbm.at[c, pl.ds(row0, RPT)])
        pltpu.sync_copy(acc_sh.at[pl.ds(row0, RPT)],
                        acc_hbm.at[c, pl.ds(row0, RPT)])

    f = pl.kernel(
        body,
        out_type=[
            jax.ShapeDtypeStruct((NC, NP, PH), jnp.float32),
            jax.ShapeDtypeStruct((NC, NP, width), jnp.float32),
        ],
        mesh=_sc_mesh(),
        scratch_types=[
            pltpu.VMEM((CH,), jnp.int32),
            pltpu.VMEM((CH, PH), jnp.float32),
            pltpu.VMEM((CH, width), jnp.float32),
            pltpu.VMEM((CH,), jnp.int32),
            pltpu.VMEM((CH, PH), jnp.float32),
            pltpu.VMEM((CH, width), jnp.float32),
            pltpu.VMEM_SHARED((NP, PH), jnp.float32),
            pltpu.VMEM_SHARED((NP, width), jnp.float32),
            pltpu.SemaphoreType.DMA,
            pltpu.SemaphoreType.DMA,
            pltpu.SemaphoreType.DMA,
            pltpu.SemaphoreType.DMA,
            pltpu.SemaphoreType.DMA,
        ],
    )
    return f(ex, msg, dst)


def _gat_edge_pass(xl, xr, edge_attr, we, src, dst, att_mat, *, heads,
                   width):
    gxl, gxr = _sc_gather(xl, xr, src, dst, width=width)
    ex, msg = _edge_tc(gxl, gxr, edge_attr, we, att_mat, heads, block_m=8000)
    den_p, acc_p = _sc_scatter(ex, msg, dst, heads=heads, width=width)
    return acc_p, den_p


def kernel(node_fts, edge_index, edge_attr, Wl1, Wr1, We1, att1, b1,
           Wl2, Wr2, We2, att2, b2):
    src = edge_index[0]
    dst = edge_index[1]

    z64 = jnp.zeros((OUT, 1), jnp.float32)
    am1 = jnp.concatenate([
        jnp.concatenate([att1[0][:, None], z64], axis=1),
        jnp.concatenate([z64, att1[1][:, None]], axis=1),
    ], axis=0)
    am2 = jnp.concatenate([att2.T, z64], axis=0)
    zp = jnp.zeros((HEADS * OUT, OUT), jnp.float32)
    Wl2p = jnp.concatenate([Wl2, zp], axis=1)
    Wr2p = jnp.concatenate([Wr2, zp], axis=1)
    We2p = jnp.concatenate([We2, jnp.zeros((D_EDGE, OUT), jnp.float32)],
                           axis=1)

    xl1, xr1 = _mm2(node_fts, Wl1, Wr1, block_m=1000)

    acc1, den1 = _gat_edge_pass(xl1, xr1, edge_attr, We1, src, dst, am1,
                                heads=HEADS, width=HEADS * OUT)

    xl2, xr2 = _combine_mm2(acc1[0, :N], acc1[1, :N], den1[0, :N],
                            den1[1, :N], b1, Wl2p, Wr2p, heads=HEADS,
                            block_m=1000)

    acc2, den2 = _gat_edge_pass(xl2, xr2, edge_attr, We2p, src, dst, am2,
                                heads=1, width=HEADS * OUT)

    return _combine_final(acc2[0, :N], acc2[1, :N], den2[0, :N],
                          den2[1, :N], b2, block_m=1000)
